# baseline (speedup 1.0000x reference)
import functools

import jax
import jax.numpy as jnp
from jax import lax
from jax.experimental import pallas as pl
from jax.experimental.pallas import tpu as pltpu
from jax.experimental.pallas import tpu_sc as plsc

N = 10000
F = 128
E = 320000
NC = 2
NS = 16
NW = NC * NS
EPW = E // NW
CB = 40
NCHUNK = EPW // CB
G = 50
NG = NCHUNK // G
NBUF = 4
NFULL = ((G - 1) // NBUF) * NBUF
RPS = 624
TAIL0 = RPS * NS
TAILN = N - TAIL0

RB = 10000


def _mm_relu_body(a_ref, w_ref, o_ref):
    s = a_ref[0] + a_ref[1]
    o_ref[...] = jnp.maximum(
        jnp.dot(s, w_ref[...], preferred_element_type=jnp.float32), 0.0)


def _mm_lsm_body(a_ref, w_ref, o_ref):
    s = jnp.dot(a_ref[0] + a_ref[1], w_ref[...],
                preferred_element_type=jnp.float32)
    m = jnp.max(s, axis=-1, keepdims=True)
    e = jnp.exp(s - m)
    o_ref[...] = s - m - jnp.log(jnp.sum(e, axis=-1, keepdims=True))


def _combine_mm(acc, W, body):
    return pl.pallas_call(
        body,
        grid=(N // RB,),
        in_specs=[pl.BlockSpec((NC, RB, F), lambda i: (0, i, 0)),
                  pl.BlockSpec((F, F), lambda i: (0, 0))],
        out_specs=pl.BlockSpec((RB, F), lambda i: (i, 0)),
        out_shape=jax.ShapeDtypeStruct((N, F), jnp.float32),
    )(acc, W)


def _sc_spmm(sup, ei4):
    mesh = plsc.VectorSubcoreMesh(core_axis_name="c", subcore_axis_name="s")

    @functools.partial(
        pl.kernel,
        out_type=jax.ShapeDtypeStruct((NC, N, F), jnp.float32),
        mesh=mesh,
        scratch_types=[
            pltpu.VMEM((2, G, CB), jnp.int32),
            pltpu.VMEM((2, G, CB), jnp.int32),
            pltpu.VMEM((NBUF, CB, F), jnp.float32),
            pltpu.VMEM_SHARED((N, F), jnp.float32),
        ] + [pltpu.SemaphoreType.DMA] * (2 * NBUF + 2),
    )
    def k(sup_hbm, ei_hbm, out_hbm,
          src_i, dst_i, rows_v, acc, *sems):
        cid = lax.axis_index("c")
        sid = lax.axis_index("s")
        wid = sid * NC + cid
        r0 = sid * RPS

        GSEMS = sems[:NBUF]
        SSEMS = sems[NBUF:2 * NBUF]
        isems = sems[2 * NBUF:]

        def idx_start(g, s):
            pltpu.async_copy(ei_hbm.at[0, wid * NG + g], src_i.at[s], isems[s])
            pltpu.async_copy(ei_hbm.at[1, wid * NG + g], dst_i.at[s], isems[s])

        def idx_wait(g, s):
            pltpu.make_async_copy(ei_hbm.at[0, wid * NG + g],
                                  src_i.at[s], isems[s]).wait()
            pltpu.make_async_copy(ei_hbm.at[1, wid * NG + g],
                                  dst_i.at[s], isems[s]).wait()

        idx_start(0, 0)
        idx_start(1, 1)

        zv = jnp.zeros((16,), jnp.float32)

        @pl.loop(0, CB)
        def _(r):
            @pl.loop(0, F, step=16)
            def _(c2):
                rows_v[0, r, pl.ds(c2, 16)] = zv

        for i in range(RPS // CB):
            pltpu.sync_copy(rows_v.at[0],
                            acc.at[pl.ds(r0 + i * CB, CB)])
        _zrem = RPS % CB
        if _zrem:
            pltpu.sync_copy(rows_v.at[0, pl.ds(0, _zrem)],
                            acc.at[pl.ds(r0 + (RPS // CB) * CB, _zrem)])

        @pl.when(sid == 0)
        def _():
            pltpu.sync_copy(rows_v.at[0, pl.ds(0, TAILN)],
                            acc.at[pl.ds(TAIL0, TAILN)])

        plsc.subcore_barrier()

        BUFS = tuple(rows_v.at[b] for b in range(NBUF))

        def g_start(s, jj, b):
            pltpu.async_copy(sup_hbm.at[src_i.at[s, jj]], BUFS[b], GSEMS[b])

        def g_wait(s, jj, b):
            pltpu.make_async_copy(sup_hbm.at[src_i.at[s, jj]],
                                  BUFS[b], GSEMS[b]).wait()

        def s_start(s, jj, b):
            pltpu.async_copy(BUFS[b], acc.at[dst_i.at[s, jj]], SSEMS[b],
                             add=True)

        def s_wait(s, jj, b):
            pltpu.make_async_copy(BUFS[b], acc.at[dst_i.at[s, jj]],
                                  SSEMS[b]).wait()

        for g in range(NG):
            s = g % 2
            idx_wait(g, s)
            for c in range(NBUF - 1):
                g_start(s, c, c)
            g_start(s, 3, 3)
            g_wait(s, 0, 0)
            s_start(s, 0, 0)

            @pl.loop(1, G - 5, step=NBUF)
            def _(jj):
                for k in range(NBUF):
                    c = jj + k
                    bc = (1 + k) % NBUF
                    s_wait(s, c - 1, k)
                    g_start(s, c + 3, k)
                    g_wait(s, c, bc)
                    s_start(s, c, bc)

            for c in (G - 5, G - 4):
                s_wait(s, c - 1, (c - 1) % NBUF)
                g_start(s, c + 3, (c + 3) % NBUF)
                g_wait(s, c, c % NBUF)
                s_start(s, c, c % NBUF)
            for c in (G - 3, G - 2, G - 1):
                s_wait(s, c - 1, (c - 1) % NBUF)
                g_wait(s, c, c % NBUF)
                s_start(s, c, c % NBUF)
            s_wait(s, G - 1, (G - 1) % NBUF)
            if g + 2 < NG:
                idx_start(g + 2, s)

        plsc.subcore_barrier()
        pltpu.sync_copy(acc.at[pl.ds(r0, RPS)],
                        out_hbm.at[cid, pl.ds(r0, RPS)])

        @pl.when(sid == 0)
        def _():
            pltpu.sync_copy(acc.at[pl.ds(TAIL0, TAILN)],
                            out_hbm.at[cid, pl.ds(TAIL0, TAILN)])

    return k(sup, ei4)


def kernel(x, edge_index, W1, W2):
    ei4 = edge_index.astype(jnp.int32).reshape(2, NW * NG, G, CB)
    a1 = _sc_spmm(x, ei4)
    h = _combine_mm(a1, W1, _mm_relu_body)
    a2 = _sc_spmm(h, ei4)
    return _combine_mm(a2, W2, _mm_lsm_body)


# speedup vs baseline: 1.0116x; 1.0116x over previous
import functools

import jax
import jax.numpy as jnp
from jax import lax
from jax.experimental import pallas as pl
from jax.experimental.pallas import tpu as pltpu
from jax.experimental.pallas import tpu_sc as plsc

N = 10000
F = 128
E = 320000
NC = 2
NS = 16
NW = NC * NS
EPW = E // NW
CB = 40
NCHUNK = EPW // CB
G = 50
NG = NCHUNK // G
NBUF = 4
NFULL = ((G - 1) // NBUF) * NBUF
RPS = 624
TAIL0 = RPS * NS
TAILN = N - TAIL0

RB = 5000


def _mm_relu_body(a_ref, w_ref, o_ref):
    s = a_ref[0] + a_ref[1]
    o_ref[...] = jnp.maximum(
        jnp.dot(s, w_ref[...], preferred_element_type=jnp.float32), 0.0)


def _mm_lsm_body(a_ref, w_ref, o_ref):
    s = jnp.dot(a_ref[0] + a_ref[1], w_ref[...],
                preferred_element_type=jnp.float32)
    m = jnp.max(s, axis=-1, keepdims=True)
    e = jnp.exp(s - m)
    o_ref[...] = s - m - jnp.log(jnp.sum(e, axis=-1, keepdims=True))


def _combine_mm(acc, W, body):
    return pl.pallas_call(
        body,
        grid=(N // RB,),
        in_specs=[pl.BlockSpec((NC, RB, F), lambda i: (0, i, 0)),
                  pl.BlockSpec((F, F), lambda i: (0, 0))],
        out_specs=pl.BlockSpec((RB, F), lambda i: (i, 0)),
        out_shape=jax.ShapeDtypeStruct((N, F), jnp.float32),
    )(acc, W)


def _sc_spmm(sup, ei4):
    mesh = plsc.VectorSubcoreMesh(core_axis_name="c", subcore_axis_name="s")

    @functools.partial(
        pl.kernel,
        out_type=jax.ShapeDtypeStruct((NC, N, F), jnp.float32),
        mesh=mesh,
        scratch_types=[
            pltpu.VMEM((2, G, CB), jnp.int32),
            pltpu.VMEM((2, G, CB), jnp.int32),
            pltpu.VMEM((NBUF, CB, F), jnp.float32),
            pltpu.VMEM_SHARED((N, F), jnp.float32),
        ] + [pltpu.SemaphoreType.DMA] * (2 * NBUF + 2),
    )
    def k(sup_hbm, ei_hbm, out_hbm,
          src_i, dst_i, rows_v, acc, *sems):
        cid = lax.axis_index("c")
        sid = lax.axis_index("s")
        wid = sid * NC + cid
        r0 = sid * RPS

        GSEMS = sems[:NBUF]
        SSEMS = sems[NBUF:2 * NBUF]
        isems = sems[2 * NBUF:]

        def idx_start(g, s):
            pltpu.async_copy(ei_hbm.at[0, wid * NG + g], src_i.at[s], isems[s])
            pltpu.async_copy(ei_hbm.at[1, wid * NG + g], dst_i.at[s], isems[s])

        def idx_wait(g, s):
            pltpu.make_async_copy(ei_hbm.at[0, wid * NG + g],
                                  src_i.at[s], isems[s]).wait()
            pltpu.make_async_copy(ei_hbm.at[1, wid * NG + g],
                                  dst_i.at[s], isems[s]).wait()

        idx_start(0, 0)
        idx_start(1, 1)

        zv = jnp.zeros((16,), jnp.float32)

        @pl.loop(0, CB)
        def _(r):
            @pl.loop(0, F, step=16)
            def _(c2):
                rows_v[0, r, pl.ds(c2, 16)] = zv

        for i in range(RPS // CB):
            pltpu.sync_copy(rows_v.at[0],
                            acc.at[pl.ds(r0 + i * CB, CB)])
        _zrem = RPS % CB
        if _zrem:
            pltpu.sync_copy(rows_v.at[0, pl.ds(0, _zrem)],
                            acc.at[pl.ds(r0 + (RPS // CB) * CB, _zrem)])

        @pl.when(sid == 0)
        def _():
            pltpu.sync_copy(rows_v.at[0, pl.ds(0, TAILN)],
                            acc.at[pl.ds(TAIL0, TAILN)])

        plsc.subcore_barrier()

        BUFS = tuple(rows_v.at[b] for b in range(NBUF))

        def g_start(s, jj, b):
            pltpu.async_copy(sup_hbm.at[src_i.at[s, jj]], BUFS[b], GSEMS[b])

        def g_wait(s, jj, b):
            pltpu.make_async_copy(sup_hbm.at[src_i.at[s, jj]],
                                  BUFS[b], GSEMS[b]).wait()

        def s_start(s, jj, b):
            pltpu.async_copy(BUFS[b], acc.at[dst_i.at[s, jj]], SSEMS[b],
                             add=True)

        def s_wait(s, jj, b):
            pltpu.make_async_copy(BUFS[b], acc.at[dst_i.at[s, jj]],
                                  SSEMS[b]).wait()

        for g in range(NG):
            s = g % 2
            idx_wait(g, s)
            for c in range(NBUF - 1):
                g_start(s, c, c)
            g_start(s, 3, 3)
            g_wait(s, 0, 0)
            s_start(s, 0, 0)

            @pl.loop(1, G - 5, step=NBUF)
            def _(jj):
                for k in range(NBUF):
                    c = jj + k
                    bc = (1 + k) % NBUF
                    s_wait(s, c - 1, k)
                    g_start(s, c + 3, k)
                    g_wait(s, c, bc)
                    s_start(s, c, bc)

            for c in (G - 5, G - 4):
                s_wait(s, c - 1, (c - 1) % NBUF)
                g_start(s, c + 3, (c + 3) % NBUF)
                g_wait(s, c, c % NBUF)
                s_start(s, c, c % NBUF)
            for c in (G - 3, G - 2, G - 1):
                s_wait(s, c - 1, (c - 1) % NBUF)
                g_wait(s, c, c % NBUF)
                s_start(s, c, c % NBUF)
            s_wait(s, G - 1, (G - 1) % NBUF)
            if g + 2 < NG:
                idx_start(g + 2, s)

        plsc.subcore_barrier()
        pltpu.sync_copy(acc.at[pl.ds(r0, RPS)],
                        out_hbm.at[cid, pl.ds(r0, RPS)])

        @pl.when(sid == 0)
        def _():
            pltpu.sync_copy(acc.at[pl.ds(TAIL0, TAILN)],
                            out_hbm.at[cid, pl.ds(TAIL0, TAILN)])

    return k(sup, ei4)


def kernel(x, edge_index, W1, W2):
    ei4 = edge_index.astype(jnp.int32).reshape(2, NW * NG, G, CB)
    a1 = _sc_spmm(x, ei4)
    h = _combine_mm(a1, W1, _mm_relu_body)
    a2 = _sc_spmm(h, ei4)
    return _combine_mm(a2, W2, _mm_lsm_body)
